# Initial kernel scaffold; baseline (speedup 1.0000x reference)
import jax
import jax.numpy as jnp
from jax.experimental import pallas as pl


def kernel(tile_continuous, tile_discrete, entity_continuous, entity_discrete, entity_N, tile_cont_w, tile_cont_b, tile_table, tile_attr_w, tile_attr_b, ent_cont_w, ent_cont_b, ent_table, ent_attr_w, ent_attr_b):
    raise NotImplementedError("write your pallas kernel here")


# speedup vs baseline: 3.8761x; 3.8761x over previous
import functools

import jax
import jax.numpy as jnp
from jax import lax
from jax.experimental import pallas as pl
from jax.experimental.pallas import tpu as pltpu
from jax.experimental.pallas import tpu_sc as plsc

H = 128
VOC = 4096
TCN, TDN = 4, 3
ECN, EDN = 13, 5
BATCH, NTT, NTE = 256, 225, 100

_TILE_SCALE = (1.0, 0.0, 0.02, 0.02)
_ENT_SCALE = (1.0, 0.0, 0.0, 0.05, 0.0, 0.02, 0.02, 0.1, 0.01, 0.1, 0.1, 0.1, 0.3)

_F32 = jnp.float32
_PREC = lax.Precision.HIGHEST

_NC, _NS = 2, 16
_NW = _NC * _NS
_TCHUNK, _TITERS = 120, 15
_ECHUNK, _EITERS = 80, 10
_TPW = _TCHUNK * _TITERS
_EPW = _ECHUNK * _EITERS
_ROWBUF = 400


def _tbl_body(tab_ref, wd_ref, out_ref):
    j = pl.program_id(0)
    w = wd_ref[pl.ds(j * H, H), :]
    out_ref[0] = jnp.dot(tab_ref[...], w, preferred_element_type=_F32,
                         precision=_PREC)


def _transform_table(table, wd, d):
    return pl.pallas_call(
        _tbl_body,
        grid=(d,),
        in_specs=[
            pl.BlockSpec((VOC, H), lambda j: (0, 0)),
            pl.BlockSpec(wd.shape, lambda j: (0, 0)),
        ],
        out_specs=pl.BlockSpec((1, VOC, H), lambda j: (j, 0, 0)),
        out_shape=jax.ShapeDtypeStruct((d, VOC, H), _F32),
    )(table, wd)


def _params_body(tcw, tcb, twc, tab, ecw, ecb, ewc, eab,
                 ut_ref, vt_ref, ue_ref, ve_ref):
    def one(cw_ref, cb_ref, wc_ref, ab_ref, u_ref, v_ref, n, scale):
        vacc = ab_ref[...]
        for a in range(n):
            w = wc_ref[pl.ds(a * H, H), :]
            u_ref[a:a + 1, :] = scale[a] * jnp.dot(
                cw_ref[a:a + 1, :], w, preferred_element_type=_F32,
                precision=_PREC)
            vacc = vacc + jnp.dot(cb_ref[a:a + 1, :], w,
                                  preferred_element_type=_F32, precision=_PREC)
        v_ref[...] = vacc

    one(tcw, tcb, twc, tab, ut_ref, vt_ref, TCN, _TILE_SCALE)
    one(ecw, ecb, ewc, eab, ue_ref, ve_ref, ECN, _ENT_SCALE)


def _sc_body(tt_hbm, it0_h, it1_h, it2_h, te_hbm, ie0_h, ie1_h, ie2_h, ie3_h,
             ie4_h, ot_hbm, oe_hbm,
             it0, it1, it2, ie0, ie1, ie2, ie3, ie4, rows, sem):
    wid = lax.axis_index("s") * _NC + lax.axis_index("c")

    def phase(t_hbm, i_hbms, o_hbm, idx_refs, chunk, iters, per_w):
        d = len(idx_refs)
        base0 = wid * per_w
        for itn in range(iters):
            base = base0 + itn * chunk
            for j in range(d):
                pltpu.sync_copy(i_hbms[j].at[pl.ds(base, chunk)], idx_refs[j])
            cps = [
                pltpu.async_copy(t_hbm.at[idx_refs[j]],
                                 rows.at[pl.ds(j * chunk, chunk), :], sem)
                for j in range(d)
            ]
            for cp in cps:
                cp.wait()

            def sum16(t, c):
                for k in range(H // 16):
                    sl = pl.ds(k * 16, 16)
                    s = rows[t, sl]
                    for j in range(1, d):
                        s = s + rows[j * chunk + t, sl]
                    rows[t, sl] = s
                return c

            lax.fori_loop(0, chunk, sum16, 0)
            pltpu.sync_copy(rows.at[pl.ds(0, chunk), :],
                            o_hbm.at[pl.ds(base, chunk), :])

    phase(tt_hbm, (it0_h, it1_h, it2_h), ot_hbm, (it0, it1, it2),
          _TCHUNK, _TITERS, _TPW)
    phase(te_hbm, (ie0_h, ie1_h, ie2_h, ie3_h, ie4_h), oe_hbm,
          (ie0, ie1, ie2, ie3, ie4), _ECHUNK, _EITERS, _EPW)


def _sc_gather(*args):
    fn = pl.kernel(
        _sc_body,
        mesh=plsc.VectorSubcoreMesh(core_axis_name="c", subcore_axis_name="s"),
        out_type=[
            jax.ShapeDtypeStruct((BATCH * NTT, H), _F32),
            jax.ShapeDtypeStruct((BATCH * NTE, H), _F32),
        ],
        scratch_types=[
        pltpu.VMEM((_TCHUNK,), jnp.int32),
        pltpu.VMEM((_TCHUNK,), jnp.int32),
        pltpu.VMEM((_TCHUNK,), jnp.int32),
        pltpu.VMEM((_ECHUNK,), jnp.int32),
        pltpu.VMEM((_ECHUNK,), jnp.int32),
        pltpu.VMEM((_ECHUNK,), jnp.int32),
        pltpu.VMEM((_ECHUNK,), jnp.int32),
        pltpu.VMEM((_ECHUNK,), jnp.int32),
            pltpu.VMEM((_ROWBUF, H), _F32),
            pltpu.SemaphoreType.DMA,
        ],
    )
    return fn(*args)


_BPG = 8


def _fin_body(gt_ref, ge_ref, ct_ref, ce_ref, ut_ref, vt_ref, ue_ref, ve_ref,
              out_ref):
    acc_t = gt_ref[...] + vt_ref[0][None, None, :]
    ct = ct_ref[...]
    for a in range(TCN):
        acc_t = acc_t + ct[:, :, a:a + 1] * ut_ref[a][None, None, :]
    acc_e = ge_ref[...] + ve_ref[0][None, None, :]
    ce = ce_ref[...]
    for a in range(ECN):
        acc_e = acc_e + ce[:, :, a:a + 1] * ue_ref[a][None, None, :]
    out_ref[...] = jnp.concatenate([acc_t, acc_e], axis=1)


def kernel(tile_continuous, tile_discrete, entity_continuous, entity_discrete,
           entity_N, tile_cont_w, tile_cont_b, tile_table, tile_attr_w,
           tile_attr_b, ent_cont_w, ent_cont_b, ent_table, ent_attr_w,
           ent_attr_b):
    del entity_N
    tt = _transform_table(tile_table, tile_attr_w[TCN * H:, :], TDN)
    te = _transform_table(ent_table, ent_attr_w[ECN * H:, :], EDN)

    ut, vt, ue, ve = pl.pallas_call(
        _params_body,
        out_shape=[
            jax.ShapeDtypeStruct((TCN, H), _F32),
            jax.ShapeDtypeStruct((1, H), _F32),
            jax.ShapeDtypeStruct((ECN, H), _F32),
            jax.ShapeDtypeStruct((1, H), _F32),
        ],
    )(tile_cont_w, tile_cont_b, tile_attr_w[:TCN * H, :],
      tile_attr_b.reshape(1, H), ent_cont_w, ent_cont_b,
      ent_attr_w[:ECN * H, :], ent_attr_b.reshape(1, H))

    it = [(tile_discrete[:, :, j].reshape(-1) + j * VOC).astype(jnp.int32)
          for j in range(TDN)]
    ie = [(entity_discrete[:, :, j].reshape(-1) + j * VOC).astype(jnp.int32)
          for j in range(EDN)]

    gt, ge = _sc_gather(tt.reshape(TDN * VOC, H), *it,
                        te.reshape(EDN * VOC, H), *ie)

    return pl.pallas_call(
        _fin_body,
        grid=(BATCH // _BPG,),
        in_specs=[
            pl.BlockSpec((_BPG, NTT, H), lambda b: (b, 0, 0)),
            pl.BlockSpec((_BPG, NTE, H), lambda b: (b, 0, 0)),
            pl.BlockSpec((_BPG, NTT, TCN), lambda b: (b, 0, 0)),
            pl.BlockSpec((_BPG, NTE, ECN), lambda b: (b, 0, 0)),
            pl.BlockSpec((TCN, H), lambda b: (0, 0)),
            pl.BlockSpec((1, H), lambda b: (0, 0)),
            pl.BlockSpec((ECN, H), lambda b: (0, 0)),
            pl.BlockSpec((1, H), lambda b: (0, 0)),
        ],
        out_specs=pl.BlockSpec((_BPG, NTT + NTE, H), lambda b: (b, 0, 0)),
        out_shape=jax.ShapeDtypeStruct((BATCH, NTT + NTE, H), _F32),
    )(gt.reshape(BATCH, NTT, H), ge.reshape(BATCH, NTE, H),
      tile_continuous, entity_continuous, ut, vt, ue, ve)


# speedup vs baseline: 4.4351x; 1.1442x over previous
import functools

import jax
import jax.numpy as jnp
from jax import lax
from jax.experimental import pallas as pl
from jax.experimental.pallas import tpu as pltpu
from jax.experimental.pallas import tpu_sc as plsc

H = 128
VOC = 4096
TCN, TDN = 4, 3
ECN, EDN = 13, 5
BATCH, NTT, NTE = 256, 225, 100

_TILE_SCALE = (1.0, 0.0, 0.02, 0.02)
_ENT_SCALE = (1.0, 0.0, 0.0, 0.05, 0.0, 0.02, 0.02, 0.1, 0.01, 0.1, 0.1, 0.1, 0.3)

_F32 = jnp.float32
_PREC = lax.Precision.HIGHEST

_NC, _NS = 2, 16
_NW = _NC * _NS
_TCHUNK, _TITERS = 120, 15
_ECHUNK, _EITERS = 80, 10
_TPW = _TCHUNK * _TITERS
_EPW = _ECHUNK * _EITERS
_ROWBUF = 400


def _tbl_body(tab_ref, wd_ref, out_ref):
    j = pl.program_id(0)
    w = wd_ref[pl.ds(j * H, H), :]
    out_ref[0] = jnp.dot(tab_ref[...], w, preferred_element_type=_F32,
                         precision=_PREC)


def _transform_table(table, wd, d):
    return pl.pallas_call(
        _tbl_body,
        grid=(d,),
        in_specs=[
            pl.BlockSpec((VOC, H), lambda j: (0, 0)),
            pl.BlockSpec(wd.shape, lambda j: (0, 0)),
        ],
        out_specs=pl.BlockSpec((1, VOC, H), lambda j: (j, 0, 0)),
        out_shape=jax.ShapeDtypeStruct((d, VOC, H), _F32),
    )(table, wd)


def _params_body(tcw, tcb, twc, tab, ecw, ecb, ewc, eab,
                 ut_ref, vt_ref, ue_ref, ve_ref):
    def one(cw_ref, cb_ref, wc_ref, ab_ref, u_ref, v_ref, n, scale):
        vacc = ab_ref[...]
        for a in range(n):
            w = wc_ref[pl.ds(a * H, H), :]
            u_ref[a:a + 1, :] = scale[a] * jnp.dot(
                cw_ref[a:a + 1, :], w, preferred_element_type=_F32,
                precision=_PREC)
            vacc = vacc + jnp.dot(cb_ref[a:a + 1, :], w,
                                  preferred_element_type=_F32, precision=_PREC)
        v_ref[...] = vacc

    one(tcw, tcb, twc, tab, ut_ref, vt_ref, TCN, _TILE_SCALE)
    one(ecw, ecb, ewc, eab, ue_ref, ve_ref, ECN, _ENT_SCALE)


def _sc_body(tt_hbm, it0_h, it1_h, it2_h, te_hbm, ie0_h, ie1_h, ie2_h, ie3_h,
             ie4_h, ot_hbm, oe_hbm,
             it0, it1, it2, ie0, ie1, ie2, ie3, ie4, rows, sem):
    wid = lax.axis_index("s") * _NC + lax.axis_index("c")

    def phase(t_hbm, i_hbms, o_hbm, idx_refs, chunk, iters, per_w):
        d = len(idx_refs)
        base0 = wid * per_w
        for itn in range(iters):
            base = base0 + itn * chunk
            for j in range(d):
                pltpu.sync_copy(i_hbms[j].at[pl.ds(base, chunk)], idx_refs[j])
            cps = [
                pltpu.async_copy(t_hbm.at[idx_refs[j]],
                                 rows.at[pl.ds(j * chunk, chunk), :], sem)
                for j in range(d)
            ]
            for cp in cps:
                cp.wait()

            def sum16(t, c):
                for k in range(H // 16):
                    sl = pl.ds(k * 16, 16)
                    s = rows[t, sl]
                    for j in range(1, d):
                        s = s + rows[j * chunk + t, sl]
                    rows[t, sl] = s
                return c

            lax.fori_loop(0, chunk, sum16, 0)
            pltpu.sync_copy(rows.at[pl.ds(0, chunk), :],
                            o_hbm.at[pl.ds(base, chunk), :])

    phase(tt_hbm, (it0_h, it1_h, it2_h), ot_hbm, (it0, it1, it2),
          _TCHUNK, _TITERS, _TPW)
    phase(te_hbm, (ie0_h, ie1_h, ie2_h, ie3_h, ie4_h), oe_hbm,
          (ie0, ie1, ie2, ie3, ie4), _ECHUNK, _EITERS, _EPW)


def _sc_gather(*args):
    fn = pl.kernel(
        _sc_body,
        mesh=plsc.VectorSubcoreMesh(core_axis_name="c", subcore_axis_name="s"),
        out_type=[
            jax.ShapeDtypeStruct((BATCH * NTT, H), _F32),
            jax.ShapeDtypeStruct((BATCH * NTE, H), _F32),
        ],
        scratch_types=[
        pltpu.VMEM((_TCHUNK,), jnp.int32),
        pltpu.VMEM((_TCHUNK,), jnp.int32),
        pltpu.VMEM((_TCHUNK,), jnp.int32),
        pltpu.VMEM((_ECHUNK,), jnp.int32),
        pltpu.VMEM((_ECHUNK,), jnp.int32),
        pltpu.VMEM((_ECHUNK,), jnp.int32),
        pltpu.VMEM((_ECHUNK,), jnp.int32),
        pltpu.VMEM((_ECHUNK,), jnp.int32),
            pltpu.VMEM((_ROWBUF, H), _F32),
            pltpu.SemaphoreType.DMA,
        ],
    )
    return fn(*args)


_BPG = 8


def _fin_body(gt_ref, ge_ref, ctt_ref, cet_ref, ut_ref, vt_ref, ue_ref, ve_ref,
              out_ref):
    dnums = (((0,), (0,)), ((), ()))
    for i in range(_BPG):
        cont_t = lax.dot_general(ctt_ref[:, i, :], ut_ref[...], dnums,
                                 preferred_element_type=_F32, precision=_PREC)
        out_ref[i, :NTT, :] = (gt_ref[pl.ds(i * NTT, NTT), :] + cont_t
                               + vt_ref[0][None, :])
        cont_e = lax.dot_general(cet_ref[:, i, :], ue_ref[...], dnums,
                                 preferred_element_type=_F32, precision=_PREC)
        out_ref[i, NTT:, :] = (ge_ref[pl.ds(i * NTE, NTE), :] + cont_e
                               + ve_ref[0][None, :])


def kernel(tile_continuous, tile_discrete, entity_continuous, entity_discrete,
           entity_N, tile_cont_w, tile_cont_b, tile_table, tile_attr_w,
           tile_attr_b, ent_cont_w, ent_cont_b, ent_table, ent_attr_w,
           ent_attr_b):
    del entity_N
    tt = _transform_table(tile_table, tile_attr_w[TCN * H:, :], TDN)
    te = _transform_table(ent_table, ent_attr_w[ECN * H:, :], EDN)

    ut, vt, ue, ve = pl.pallas_call(
        _params_body,
        out_shape=[
            jax.ShapeDtypeStruct((TCN, H), _F32),
            jax.ShapeDtypeStruct((1, H), _F32),
            jax.ShapeDtypeStruct((ECN, H), _F32),
            jax.ShapeDtypeStruct((1, H), _F32),
        ],
    )(tile_cont_w, tile_cont_b, tile_attr_w[:TCN * H, :],
      tile_attr_b.reshape(1, H), ent_cont_w, ent_cont_b,
      ent_attr_w[:ECN * H, :], ent_attr_b.reshape(1, H))

    it = [(tile_discrete[:, :, j].reshape(-1) + j * VOC).astype(jnp.int32)
          for j in range(TDN)]
    ie = [(entity_discrete[:, :, j].reshape(-1) + j * VOC).astype(jnp.int32)
          for j in range(EDN)]

    gt, ge = _sc_gather(tt.reshape(TDN * VOC, H), *it,
                        te.reshape(EDN * VOC, H), *ie)

    ctt = jnp.transpose(tile_continuous, (2, 0, 1))
    cet = jnp.transpose(entity_continuous, (2, 0, 1))
    return pl.pallas_call(
        _fin_body,
        grid=(BATCH // _BPG,),
        in_specs=[
            pl.BlockSpec((_BPG * NTT, H), lambda b: (b, 0)),
            pl.BlockSpec((_BPG * NTE, H), lambda b: (b, 0)),
            pl.BlockSpec((TCN, _BPG, NTT), lambda b: (0, b, 0)),
            pl.BlockSpec((ECN, _BPG, NTE), lambda b: (0, b, 0)),
            pl.BlockSpec((TCN, H), lambda b: (0, 0)),
            pl.BlockSpec((1, H), lambda b: (0, 0)),
            pl.BlockSpec((ECN, H), lambda b: (0, 0)),
            pl.BlockSpec((1, H), lambda b: (0, 0)),
        ],
        out_specs=pl.BlockSpec((_BPG, NTT + NTE, H), lambda b: (b, 0, 0)),
        out_shape=jax.ShapeDtypeStruct((BATCH, NTT + NTE, H), _F32),
    )(gt, ge, ctt, cet, ut, vt, ue, ve)


# speedup vs baseline: 6.2115x; 1.4005x over previous
import functools

import jax
import jax.numpy as jnp
from jax import lax
from jax.experimental import pallas as pl
from jax.experimental.pallas import tpu as pltpu
from jax.experimental.pallas import tpu_sc as plsc

H = 128
VOC = 4096
TCN, TDN = 4, 3
ECN, EDN = 13, 5
BATCH, NTT, NTE = 256, 225, 100

_TILE_SCALE = (1.0, 0.0, 0.02, 0.02)
_ENT_SCALE = (1.0, 0.0, 0.0, 0.05, 0.0, 0.02, 0.02, 0.1, 0.01, 0.1, 0.1, 0.1, 0.3)

_F32 = jnp.float32
_PREC = lax.Precision.HIGHEST

_NC, _NS = 2, 16
_NW = _NC * _NS
_TCHUNK, _TITERS = 120, 15
_ECHUNK, _EITERS = 80, 10
_TPW = _TCHUNK * _TITERS
_EPW = _ECHUNK * _EITERS
_ROWBUF = 400


def _tbl_body(tab_ref, wd_ref, out_ref):
    j = pl.program_id(0)
    w = wd_ref[pl.ds(j * H, H), :]
    out_ref[0] = jnp.dot(tab_ref[...], w, preferred_element_type=_F32)


def _transform_table(table, wd, d):
    return pl.pallas_call(
        _tbl_body,
        grid=(d,),
        in_specs=[
            pl.BlockSpec((VOC, H), lambda j: (0, 0)),
            pl.BlockSpec(wd.shape, lambda j: (0, 0)),
        ],
        out_specs=pl.BlockSpec((1, VOC, H), lambda j: (j, 0, 0)),
        out_shape=jax.ShapeDtypeStruct((d, VOC, H), _F32),
    )(table, wd)


def _params_body(tcw, tcb, twc, tab, ecw, ecb, ewc, eab,
                 ut_ref, vt_ref, ue_ref, ve_ref):
    def one(cw_ref, cb_ref, wc_ref, ab_ref, u_ref, v_ref, n, scale):
        vacc = ab_ref[...]
        for a in range(n):
            w = wc_ref[pl.ds(a * H, H), :]
            u_ref[a:a + 1, :] = scale[a] * jnp.dot(
                cw_ref[a:a + 1, :], w, preferred_element_type=_F32,
                precision=_PREC)
            vacc = vacc + jnp.dot(cb_ref[a:a + 1, :], w,
                                  preferred_element_type=_F32, precision=_PREC)
        v_ref[...] = vacc

    one(tcw, tcb, twc, tab, ut_ref, vt_ref, TCN, _TILE_SCALE)
    one(ecw, ecb, ewc, eab, ue_ref, ve_ref, ECN, _ENT_SCALE)


_HALF = 400


def _sc_body(*refs):
    (tt_hbm, it0_h, it1_h, it2_h, te_hbm, ie0_h, ie1_h, ie2_h, ie3_h, ie4_h,
     ot_hbm, oe_hbm,
     ti00, ti01, ti02, ti10, ti11, ti12,
     ei00, ei01, ei02, ei03, ei04, ei10, ei11, ei12, ei13, ei14,
     rows, isem0, isem1, gsem0, gsem1, osem0, osem1) = refs
    wid = lax.axis_index("s") * _NC + lax.axis_index("c")
    isems, gsems, osems = (isem0, isem1), (gsem0, gsem1), (osem0, osem1)

    def phase(t_hbm, i_hbms, o_hbm, idxp, chunk, iters, per_w):
        d = len(i_hbms)
        base0 = wid * per_w
        icps, gcps, ocps = {}, {}, {}

        def fire_idx(n):
            p = n % 2
            icps[n] = [pltpu.async_copy(
                i_hbms[j].at[pl.ds(base0 + n * chunk, chunk)], idxp[p][j],
                isems[p]) for j in range(d)]

        def fire_gather(n):
            p = n % 2
            gcps[n] = [pltpu.async_copy(
                t_hbm.at[idxp[p][j]],
                rows.at[pl.ds(p * _HALF + j * chunk, chunk), :], gsems[p])
                for j in range(d)]

        fire_idx(0)
        for cp in icps[0]:
            cp.wait()
        fire_gather(0)
        if iters > 1:
            fire_idx(1)
        for n in range(iters):
            p = n % 2
            if n + 1 < iters:
                for cp in icps[n + 1]:
                    cp.wait()
                if n - 1 >= 0:
                    for cp in ocps[n - 1]:
                        cp.wait()
                fire_gather(n + 1)
            for cp in gcps[n]:
                cp.wait()

            def sum16(t, c):
                for k in range(H // 16):
                    sl = pl.ds(k * 16, 16)
                    s = rows[p * _HALF + t, sl]
                    for j in range(1, d):
                        s = s + rows[p * _HALF + j * chunk + t, sl]
                    rows[p * _HALF + t, sl] = s
                return c

            lax.fori_loop(0, chunk, sum16, 0)
            ocps[n] = [pltpu.async_copy(
                rows.at[pl.ds(p * _HALF, chunk), :],
                o_hbm.at[pl.ds(base0 + n * chunk, chunk), :], osems[p])]
            if n + 2 < iters:
                fire_idx(n + 2)
        for n in (iters - 2, iters - 1):
            if n >= 0:
                for cp in ocps[n]:
                    cp.wait()

    phase(tt_hbm, (it0_h, it1_h, it2_h), ot_hbm,
          ((ti00, ti01, ti02), (ti10, ti11, ti12)),
          _TCHUNK, _TITERS, _TPW)
    phase(te_hbm, (ie0_h, ie1_h, ie2_h, ie3_h, ie4_h), oe_hbm,
          ((ei00, ei01, ei02, ei03, ei04), (ei10, ei11, ei12, ei13, ei14)),
          _ECHUNK, _EITERS, _EPW)


def _sc_gather(*args):
    fn = pl.kernel(
        _sc_body,
        mesh=plsc.VectorSubcoreMesh(core_axis_name="c", subcore_axis_name="s"),
        out_type=[
            jax.ShapeDtypeStruct((BATCH * NTT, H), _F32),
            jax.ShapeDtypeStruct((BATCH * NTE, H), _F32),
        ],
        scratch_types=(
            [pltpu.VMEM((_TCHUNK,), jnp.int32)] * (2 * TDN)
            + [pltpu.VMEM((_ECHUNK,), jnp.int32)] * (2 * EDN)
            + [pltpu.VMEM((2 * _HALF, H), _F32)]
            + [pltpu.SemaphoreType.DMA] * 6
        ),
    )
    return fn(*args)


_BPG = 8


def _fin_body(gt_ref, ge_ref, ctt_ref, cet_ref, ut_ref, vt_ref, ue_ref, ve_ref,
              out_ref):
    dnums = (((0,), (0,)), ((), ()))
    for i in range(_BPG):
        cont_t = lax.dot_general(ctt_ref[:, i, :], ut_ref[...], dnums,
                                 preferred_element_type=_F32, precision=_PREC)
        out_ref[i, :NTT, :] = (gt_ref[pl.ds(i * NTT, NTT), :] + cont_t
                               + vt_ref[0][None, :])
        cont_e = lax.dot_general(cet_ref[:, i, :], ue_ref[...], dnums,
                                 preferred_element_type=_F32, precision=_PREC)
        out_ref[i, NTT:, :] = (ge_ref[pl.ds(i * NTE, NTE), :] + cont_e
                               + ve_ref[0][None, :])


def kernel(tile_continuous, tile_discrete, entity_continuous, entity_discrete,
           entity_N, tile_cont_w, tile_cont_b, tile_table, tile_attr_w,
           tile_attr_b, ent_cont_w, ent_cont_b, ent_table, ent_attr_w,
           ent_attr_b):
    del entity_N
    tt = _transform_table(tile_table, tile_attr_w[TCN * H:, :], TDN)
    te = _transform_table(ent_table, ent_attr_w[ECN * H:, :], EDN)

    ut, vt, ue, ve = pl.pallas_call(
        _params_body,
        out_shape=[
            jax.ShapeDtypeStruct((TCN, H), _F32),
            jax.ShapeDtypeStruct((1, H), _F32),
            jax.ShapeDtypeStruct((ECN, H), _F32),
            jax.ShapeDtypeStruct((1, H), _F32),
        ],
    )(tile_cont_w, tile_cont_b, tile_attr_w[:TCN * H, :],
      tile_attr_b.reshape(1, H), ent_cont_w, ent_cont_b,
      ent_attr_w[:ECN * H, :], ent_attr_b.reshape(1, H))

    it = [(tile_discrete[:, :, j].reshape(-1) + j * VOC).astype(jnp.int32)
          for j in range(TDN)]
    ie = [(entity_discrete[:, :, j].reshape(-1) + j * VOC).astype(jnp.int32)
          for j in range(EDN)]

    gt, ge = _sc_gather(tt.reshape(TDN * VOC, H), *it,
                        te.reshape(EDN * VOC, H), *ie)

    ctt = jnp.transpose(tile_continuous, (2, 0, 1))
    cet = jnp.transpose(entity_continuous, (2, 0, 1))
    return pl.pallas_call(
        _fin_body,
        grid=(BATCH // _BPG,),
        in_specs=[
            pl.BlockSpec((_BPG * NTT, H), lambda b: (b, 0)),
            pl.BlockSpec((_BPG * NTE, H), lambda b: (b, 0)),
            pl.BlockSpec((TCN, _BPG, NTT), lambda b: (0, b, 0)),
            pl.BlockSpec((ECN, _BPG, NTE), lambda b: (0, b, 0)),
            pl.BlockSpec((TCN, H), lambda b: (0, 0)),
            pl.BlockSpec((1, H), lambda b: (0, 0)),
            pl.BlockSpec((ECN, H), lambda b: (0, 0)),
            pl.BlockSpec((1, H), lambda b: (0, 0)),
        ],
        out_specs=pl.BlockSpec((_BPG, NTT + NTE, H), lambda b: (b, 0, 0)),
        out_shape=jax.ShapeDtypeStruct((BATCH, NTT + NTE, H), _F32),
    )(gt, ge, ctt, cet, ut, vt, ue, ve)


# speedup vs baseline: 7.2266x; 1.1634x over previous
import functools

import jax
import jax.numpy as jnp
from jax import lax
from jax.experimental import pallas as pl
from jax.experimental.pallas import tpu as pltpu
from jax.experimental.pallas import tpu_sc as plsc

H = 128
VOC = 4096
TCN, TDN = 4, 3
ECN, EDN = 13, 5
BATCH, NTT, NTE = 256, 225, 100

_TILE_SCALE = (1.0, 0.0, 0.02, 0.02)
_ENT_SCALE = (1.0, 0.0, 0.0, 0.05, 0.0, 0.02, 0.02, 0.1, 0.01, 0.1, 0.1, 0.1, 0.3)

_F32 = jnp.float32
_PREC = lax.Precision.HIGHEST

_NC, _NS = 2, 16
_NW = _NC * _NS
_TCHUNK, _TITERS = 120, 15
_ECHUNK, _EITERS = 80, 10
_TPW = _TCHUNK * _TITERS
_EPW = _ECHUNK * _EITERS
_ROWBUF = 400


def _tbl_body(tab_ref, wd_ref, out_ref):
    j = pl.program_id(0)
    w = wd_ref[pl.ds(j * H, H), :]
    out_ref[0] = jnp.dot(tab_ref[...], w, preferred_element_type=_F32)


def _transform_table(table, wd, d):
    return pl.pallas_call(
        _tbl_body,
        grid=(d,),
        in_specs=[
            pl.BlockSpec((VOC, H), lambda j: (0, 0)),
            pl.BlockSpec(wd.shape, lambda j: (0, 0)),
        ],
        out_specs=pl.BlockSpec((1, VOC, H), lambda j: (j, 0, 0)),
        out_shape=jax.ShapeDtypeStruct((d, VOC, H), _F32),
    )(table, wd)


def _params_body(tcw, tcb, twc, tab, ecw, ecb, ewc, eab,
                 ut_ref, vt_ref, ue_ref, ve_ref):
    def one(cw_ref, cb_ref, wc_ref, ab_ref, u_ref, v_ref, n, scale):
        vacc = ab_ref[...]
        for a in range(n):
            w = wc_ref[pl.ds(a * H, H), :]
            u_ref[a:a + 1, :] = scale[a] * jnp.dot(
                cw_ref[a:a + 1, :], w, preferred_element_type=_F32)
            vacc = vacc + jnp.dot(cb_ref[a:a + 1, :], w,
                                  preferred_element_type=_F32)
        v_ref[...] = vacc

    one(tcw, tcb, twc, tab, ut_ref, vt_ref, TCN, _TILE_SCALE)
    one(ecw, ecb, ewc, eab, ue_ref, ve_ref, ECN, _ENT_SCALE)


_HALF = 400


def _sc_body(*refs):
    (tt_hbm, it0_h, it1_h, it2_h, te_hbm, ie0_h, ie1_h, ie2_h, ie3_h, ie4_h,
     ot_hbm, oe_hbm,
     ti00, ti01, ti02, ti10, ti11, ti12,
     ei00, ei01, ei02, ei03, ei04, ei10, ei11, ei12, ei13, ei14,
     rows, isem0, isem1, gsem0, gsem1, osem0, osem1) = refs
    wid = lax.axis_index("s") * _NC + lax.axis_index("c")
    isems, gsems, osems = (isem0, isem1), (gsem0, gsem1), (osem0, osem1)

    def phase(t_hbm, i_hbms, o_hbm, idxp, chunk, iters, per_w):
        d = len(i_hbms)
        base0 = wid * per_w
        icps, gcps, ocps = {}, {}, {}

        def fire_idx(n):
            p = n % 2
            icps[n] = [pltpu.async_copy(
                i_hbms[j].at[pl.ds(base0 + n * chunk, chunk)], idxp[p][j],
                isems[p]) for j in range(d)]

        def fire_gather(n):
            p = n % 2
            gcps[n] = [pltpu.async_copy(
                t_hbm.at[idxp[p][j]],
                rows.at[pl.ds(p * _HALF + j * chunk, chunk), :], gsems[p])
                for j in range(d)]

        fire_idx(0)
        for cp in icps[0]:
            cp.wait()
        fire_gather(0)
        if iters > 1:
            fire_idx(1)
        for n in range(iters):
            p = n % 2
            if n + 1 < iters:
                for cp in icps[n + 1]:
                    cp.wait()
                if n - 1 >= 0:
                    for cp in ocps[n - 1]:
                        cp.wait()
                fire_gather(n + 1)
            for cp in gcps[n]:
                cp.wait()

            def sum16(t, c):
                for k in range(H // 16):
                    sl = pl.ds(k * 16, 16)
                    s = rows[p * _HALF + t, sl]
                    for j in range(1, d):
                        s = s + rows[p * _HALF + j * chunk + t, sl]
                    rows[p * _HALF + t, sl] = s
                return c

            lax.fori_loop(0, chunk, sum16, 0)
            ocps[n] = [pltpu.async_copy(
                rows.at[pl.ds(p * _HALF, chunk), :],
                o_hbm.at[pl.ds(base0 + n * chunk, chunk), :], osems[p])]
            if n + 2 < iters:
                fire_idx(n + 2)
        for n in (iters - 2, iters - 1):
            if n >= 0:
                for cp in ocps[n]:
                    cp.wait()

    phase(tt_hbm, (it0_h, it1_h, it2_h), ot_hbm,
          ((ti00, ti01, ti02), (ti10, ti11, ti12)),
          _TCHUNK, _TITERS, _TPW)
    phase(te_hbm, (ie0_h, ie1_h, ie2_h, ie3_h, ie4_h), oe_hbm,
          ((ei00, ei01, ei02, ei03, ei04), (ei10, ei11, ei12, ei13, ei14)),
          _ECHUNK, _EITERS, _EPW)


def _sc_gather(*args):
    fn = pl.kernel(
        _sc_body,
        mesh=plsc.VectorSubcoreMesh(core_axis_name="c", subcore_axis_name="s"),
        out_type=[
            jax.ShapeDtypeStruct((BATCH * NTT, H), _F32),
            jax.ShapeDtypeStruct((BATCH * NTE, H), _F32),
        ],
        scratch_types=(
            [pltpu.VMEM((_TCHUNK,), jnp.int32)] * (2 * TDN)
            + [pltpu.VMEM((_ECHUNK,), jnp.int32)] * (2 * EDN)
            + [pltpu.VMEM((2 * _HALF, H), _F32)]
            + [pltpu.SemaphoreType.DMA] * 6
        ),
    )
    return fn(*args)


_BPG = 8


def _fin_body(gt_ref, ge_ref, ctt_ref, cet_ref, ut_ref, vt_ref, ue_ref, ve_ref,
              out_ref):
    dnums = (((0,), (0,)), ((), ()))
    ctt = ctt_ref[...].reshape(TCN, _BPG * NTT)
    acc_t = gt_ref[...] + lax.dot_general(
        ctt, ut_ref[...], dnums, preferred_element_type=_F32
    ) + vt_ref[0][None, :]
    cet = cet_ref[...].reshape(ECN, _BPG * NTE)
    acc_e = ge_ref[...] + lax.dot_general(
        cet, ue_ref[...], dnums, preferred_element_type=_F32
    ) + ve_ref[0][None, :]
    for i in range(_BPG):
        out_ref[i, :NTT, :] = acc_t[i * NTT:(i + 1) * NTT, :]
        out_ref[i, NTT:, :] = acc_e[i * NTE:(i + 1) * NTE, :]


def kernel(tile_continuous, tile_discrete, entity_continuous, entity_discrete,
           entity_N, tile_cont_w, tile_cont_b, tile_table, tile_attr_w,
           tile_attr_b, ent_cont_w, ent_cont_b, ent_table, ent_attr_w,
           ent_attr_b):
    del entity_N
    tt = _transform_table(tile_table, tile_attr_w[TCN * H:, :], TDN)
    te = _transform_table(ent_table, ent_attr_w[ECN * H:, :], EDN)

    it = [(tile_discrete[:, :, j].reshape(-1) + j * VOC).astype(jnp.int32)
          for j in range(TDN)]
    ie = [(entity_discrete[:, :, j].reshape(-1) + j * VOC).astype(jnp.int32)
          for j in range(EDN)]

    gt, ge = _sc_gather(tt.reshape(TDN * VOC, H), *it,
                        te.reshape(EDN * VOC, H), *ie)

    ut, vt, ue, ve = pl.pallas_call(
        _params_body,
        out_shape=[
            jax.ShapeDtypeStruct((TCN, H), _F32),
            jax.ShapeDtypeStruct((1, H), _F32),
            jax.ShapeDtypeStruct((ECN, H), _F32),
            jax.ShapeDtypeStruct((1, H), _F32),
        ],
    )(tile_cont_w, tile_cont_b, tile_attr_w[:TCN * H, :],
      tile_attr_b.reshape(1, H), ent_cont_w, ent_cont_b,
      ent_attr_w[:ECN * H, :], ent_attr_b.reshape(1, H))

    ctt = jnp.transpose(tile_continuous, (2, 0, 1))
    cet = jnp.transpose(entity_continuous, (2, 0, 1))
    return pl.pallas_call(
        _fin_body,
        grid=(BATCH // _BPG,),
        in_specs=[
            pl.BlockSpec((_BPG * NTT, H), lambda b: (b, 0)),
            pl.BlockSpec((_BPG * NTE, H), lambda b: (b, 0)),
            pl.BlockSpec((TCN, _BPG, NTT), lambda b: (0, b, 0)),
            pl.BlockSpec((ECN, _BPG, NTE), lambda b: (0, b, 0)),
            pl.BlockSpec((TCN, H), lambda b: (0, 0)),
            pl.BlockSpec((1, H), lambda b: (0, 0)),
            pl.BlockSpec((ECN, H), lambda b: (0, 0)),
            pl.BlockSpec((1, H), lambda b: (0, 0)),
        ],
        out_specs=pl.BlockSpec((_BPG, NTT + NTE, H), lambda b: (b, 0, 0)),
        out_shape=jax.ShapeDtypeStruct((BATCH, NTT + NTE, H), _F32),
    )(gt, ge, ctt, cet, ut, vt, ue, ve)


# speedup vs baseline: 9.4969x; 1.3142x over previous
import functools

import jax
import jax.numpy as jnp
from jax import lax
from jax.experimental import pallas as pl
from jax.experimental.pallas import tpu as pltpu
from jax.experimental.pallas import tpu_sc as plsc

H = 128
VOC = 4096
TCN, TDN = 4, 3
ECN, EDN = 13, 5
BATCH, NTT, NTE = 256, 225, 100

_TILE_SCALE = (1.0, 0.0, 0.02, 0.02)
_ENT_SCALE = (1.0, 0.0, 0.0, 0.05, 0.0, 0.02, 0.02, 0.1, 0.01, 0.1, 0.1, 0.1, 0.3)

_F32 = jnp.float32
_BF16 = jnp.bfloat16
_PREC = lax.Precision.HIGHEST

_NC, _NS = 2, 16
_NW = _NC * _NS
_TCHUNK, _TITERS = 120, 15
_ECHUNK, _EITERS = 80, 10
_TPW = _TCHUNK * _TITERS
_EPW = _ECHUNK * _EITERS
_ROWBUF = 400


def _tbl_body(tab_ref, wd_ref, out_ref):
    j = pl.program_id(0)
    w = wd_ref[pl.ds(j * H, H), :]
    out_ref[0] = jnp.dot(tab_ref[...], w, preferred_element_type=_F32)


def _transform_table(table, wd, d):
    return pl.pallas_call(
        _tbl_body,
        grid=(d,),
        in_specs=[
            pl.BlockSpec((VOC, H), lambda j: (0, 0)),
            pl.BlockSpec(wd.shape, lambda j: (0, 0)),
        ],
        out_specs=pl.BlockSpec((1, VOC, H), lambda j: (j, 0, 0)),
        out_shape=jax.ShapeDtypeStruct((d, VOC, H), _F32),
    )(table, wd)


def _params_body(tcw, tcb, twc, tab, ecw, ecb, ewc, eab,
                 ut_ref, vt_ref, ue_ref, ve_ref):
    def one(cw_ref, cb_ref, wc_ref, ab_ref, u_ref, v_ref, n, scale):
        vacc = ab_ref[...]
        for a in range(n):
            w = wc_ref[pl.ds(a * H, H), :]
            u_ref[a:a + 1, :] = scale[a] * jnp.dot(
                cw_ref[a:a + 1, :], w, preferred_element_type=_F32)
            vacc = vacc + jnp.dot(cb_ref[a:a + 1, :], w,
                                  preferred_element_type=_F32)
        v_ref[...] = vacc

    one(tcw, tcb, twc, tab, ut_ref, vt_ref, TCN, _TILE_SCALE)
    one(ecw, ecb, ewc, eab, ue_ref, ve_ref, ECN, _ENT_SCALE)


_HALF = 400


def _sc_body(*refs):
    (tt_hbm, it0_h, it1_h, it2_h, te_hbm, ie0_h, ie1_h, ie2_h, ie3_h, ie4_h,
     ot_hbm, oe_hbm,
     ti00, ti01, ti02, ti10, ti11, ti12,
     ei00, ei01, ei02, ei03, ei04, ei10, ei11, ei12, ei13, ei14,
     rows, isem0, isem1, gsem0, gsem1, osem0, osem1) = refs
    wid = lax.axis_index("s") * _NC + lax.axis_index("c")
    isems, gsems, osems = (isem0, isem1), (gsem0, gsem1), (osem0, osem1)

    def phase(t_hbm, i_hbms, o_hbm, idxp, chunk, iters, per_w):
        d = len(i_hbms)
        base0 = wid * per_w
        icps, gcps, ocps = {}, {}, {}

        def fire_idx(n):
            p = n % 2
            icps[n] = [pltpu.async_copy(
                i_hbms[j].at[pl.ds(base0 + n * chunk, chunk)], idxp[p][j],
                isems[p]) for j in range(d)]

        def fire_gather(n):
            p = n % 2
            gcps[n] = [pltpu.async_copy(
                t_hbm.at[idxp[p][j]],
                rows.at[pl.ds(p * _HALF + j * chunk, chunk), :], gsems[p])
                for j in range(d)]

        fire_idx(0)
        for cp in icps[0]:
            cp.wait()
        fire_gather(0)
        if iters > 1:
            fire_idx(1)
        for n in range(iters):
            p = n % 2
            if n + 1 < iters:
                for cp in icps[n + 1]:
                    cp.wait()
                if n - 1 >= 0:
                    for cp in ocps[n - 1]:
                        cp.wait()
                fire_gather(n + 1)
            for cp in gcps[n]:
                cp.wait()

            def sum16(t, c):
                for k in range(H // 16):
                    sl = pl.ds(k * 16, 16)
                    s = rows[p * _HALF + t, sl]
                    for j in range(1, d):
                        s = s + rows[p * _HALF + j * chunk + t, sl]
                    rows[p * _HALF + t, sl] = s
                return c

            lax.fori_loop(0, chunk, sum16, 0)
            ocps[n] = [pltpu.async_copy(
                rows.at[pl.ds(p * _HALF, chunk), :],
                o_hbm.at[pl.ds(base0 + n * chunk, chunk), :], osems[p])]
            if n + 2 < iters:
                fire_idx(n + 2)
        for n in (iters - 2, iters - 1):
            if n >= 0:
                for cp in ocps[n]:
                    cp.wait()

    phase(tt_hbm, (it0_h, it1_h, it2_h), ot_hbm,
          ((ti00, ti01, ti02), (ti10, ti11, ti12)),
          _TCHUNK, _TITERS, _TPW)
    phase(te_hbm, (ie0_h, ie1_h, ie2_h, ie3_h, ie4_h), oe_hbm,
          ((ei00, ei01, ei02, ei03, ei04), (ei10, ei11, ei12, ei13, ei14)),
          _ECHUNK, _EITERS, _EPW)


def _sc_gather(*args):
    fn = pl.kernel(
        _sc_body,
        mesh=plsc.VectorSubcoreMesh(core_axis_name="c", subcore_axis_name="s"),
        out_type=[
            jax.ShapeDtypeStruct((BATCH * NTT, H), _F32),
            jax.ShapeDtypeStruct((BATCH * NTE, H), _F32),
        ],
        scratch_types=(
            [pltpu.VMEM((_TCHUNK,), jnp.int32)] * (2 * TDN)
            + [pltpu.VMEM((_ECHUNK,), jnp.int32)] * (2 * EDN)
            + [pltpu.VMEM((2 * _HALF, H), _F32)]
            + [pltpu.SemaphoreType.DMA] * 6
        ),
    )
    return fn(*args)


_TOKT = 25
_NTB = NTT // _TOKT
_FGRID = (NTT + NTE) // _TOKT
_FROWS = _TOKT * BATCH


def _fin_body(gt_ref, ge_ref, ctt_ref, cet_ref, ut_ref, vt_ref, ue_ref,
              ve_ref, out_ref):
    pid = pl.program_id(0)
    dnums = (((0,), (0,)), ((), ()))

    @pl.when(pid < _NTB)
    def _tile():
        out_ref[...] = gt_ref[...] + lax.dot_general(
            ctt_ref[...], ut_ref[...], dnums, preferred_element_type=_F32
        ) + vt_ref[0][None, :]

    @pl.when(pid >= _NTB)
    def _ent():
        out_ref[...] = ge_ref[...] + lax.dot_general(
            cet_ref[...], ue_ref[...], dnums, preferred_element_type=_F32
        ) + ve_ref[0][None, :]


def kernel(tile_continuous, tile_discrete, entity_continuous, entity_discrete,
           entity_N, tile_cont_w, tile_cont_b, tile_table, tile_attr_w,
           tile_attr_b, ent_cont_w, ent_cont_b, ent_table, ent_attr_w,
           ent_attr_b):
    del entity_N
    tt = _transform_table(tile_table, tile_attr_w[TCN * H:, :], TDN)
    te = _transform_table(ent_table, ent_attr_w[ECN * H:, :], EDN)

    it = [(tile_discrete[:, :, j].T.reshape(-1) + j * VOC).astype(jnp.int32)
          for j in range(TDN)]
    ie = [(entity_discrete[:, :, j].T.reshape(-1) + j * VOC).astype(jnp.int32)
          for j in range(EDN)]

    gt, ge = _sc_gather(tt.reshape(TDN * VOC, H), *it,
                        te.reshape(EDN * VOC, H), *ie)

    ut, vt, ue, ve = pl.pallas_call(
        _params_body,
        out_shape=[
            jax.ShapeDtypeStruct((TCN, H), _F32),
            jax.ShapeDtypeStruct((1, H), _F32),
            jax.ShapeDtypeStruct((ECN, H), _F32),
            jax.ShapeDtypeStruct((1, H), _F32),
        ],
    )(tile_cont_w, tile_cont_b, tile_attr_w[:TCN * H, :],
      tile_attr_b.reshape(1, H), ent_cont_w, ent_cont_b,
      ent_attr_w[:ECN * H, :], ent_attr_b.reshape(1, H))

    ctt = jnp.transpose(tile_continuous, (2, 1, 0)).reshape(TCN, NTT * BATCH)
    cet = jnp.transpose(entity_continuous, (2, 1, 0)).reshape(ECN, NTE * BATCH)
    ntb = _NTB
    out_flat = pl.pallas_call(
        _fin_body,
        grid=(_FGRID,),
        in_specs=[
            pl.BlockSpec((_FROWS, H), lambda j: (jnp.minimum(j, ntb - 1), 0)),
            pl.BlockSpec((_FROWS, H), lambda j: (jnp.maximum(j - ntb, 0), 0)),
            pl.BlockSpec((TCN, _FROWS),
                         lambda j: (0, jnp.minimum(j, ntb - 1))),
            pl.BlockSpec((ECN, _FROWS),
                         lambda j: (0, jnp.maximum(j - ntb, 0))),
            pl.BlockSpec((TCN, H), lambda j: (0, 0)),
            pl.BlockSpec((1, H), lambda j: (0, 0)),
            pl.BlockSpec((ECN, H), lambda j: (0, 0)),
            pl.BlockSpec((1, H), lambda j: (0, 0)),
        ],
        out_specs=pl.BlockSpec((_FROWS, H), lambda j: (j, 0)),
        out_shape=jax.ShapeDtypeStruct(((NTT + NTE) * BATCH, H), _F32),
    )(gt, ge, ctt, cet, ut, vt, ue, ve)
    return jnp.transpose(out_flat.reshape(NTT + NTE, BATCH, H), (1, 0, 2))


# speedup vs baseline: 9.5125x; 1.0016x over previous
import jax
import jax.numpy as jnp
from jax import lax
from jax.experimental import pallas as pl
from jax.experimental.pallas import tpu as pltpu
from jax.experimental.pallas import tpu_sc as plsc

H = 128
VOC = 4096
TCN, TDN = 4, 3
ECN, EDN = 13, 5
BATCH, NTT, NTE = 256, 225, 100

_TILE_SCALE = (1.0, 0.0, 0.02, 0.02)
_ENT_SCALE = (1.0, 0.0, 0.0, 0.05, 0.0, 0.02, 0.02, 0.1, 0.01, 0.1, 0.1, 0.1, 0.3)

_F32 = jnp.float32
_BF16 = jnp.bfloat16

_NC, _NS = 2, 16
_NW = _NC * _NS
_TCHUNK, _TITERS = 120, 15
_ECHUNK, _EITERS = 80, 10
_TPW = _TCHUNK * _TITERS
_EPW = _ECHUNK * _EITERS


def _tbl_body(tab_ref, wd_ref, out_ref):
    j = pl.program_id(0)
    w = wd_ref[pl.ds(j * H, H), :]
    out_ref[0] = jnp.dot(tab_ref[...], w, preferred_element_type=_F32)


def _transform_table(table, wd, d):
    return pl.pallas_call(
        _tbl_body,
        grid=(d,),
        in_specs=[
            pl.BlockSpec((VOC, H), lambda j: (0, 0)),
            pl.BlockSpec(wd.shape, lambda j: (0, 0)),
        ],
        out_specs=pl.BlockSpec((1, VOC, H), lambda j: (j, 0, 0)),
        out_shape=jax.ShapeDtypeStruct((d, VOC, H), _F32),
    )(table, wd)


def _params_body(tcw, tcb, twc, tab, ecw, ecb, ewc, eab,
                 ut_ref, vt_ref, ue_ref, ve_ref):
    def one(cw_ref, cb_ref, wc_ref, ab_ref, u_ref, v_ref, n, scale):
        vacc = ab_ref[...]
        for a in range(n):
            w = wc_ref[pl.ds(a * H, H), :]
            u_ref[a:a + 1, :] = scale[a] * jnp.dot(
                cw_ref[a:a + 1, :], w, preferred_element_type=_F32)
            vacc = vacc + jnp.dot(cb_ref[a:a + 1, :], w,
                                  preferred_element_type=_F32)
        v_ref[...] = vacc

    one(tcw, tcb, twc, tab, ut_ref, vt_ref, TCN, _TILE_SCALE)
    one(ecw, ecb, ewc, eab, ue_ref, ve_ref, ECN, _ENT_SCALE)


_HALF = 400


def _sc_body(*refs):
    (tt_hbm, it0_h, it1_h, it2_h, te_hbm, ie0_h, ie1_h, ie2_h, ie3_h, ie4_h,
     ot_hbm, oe_hbm,
     ti00, ti01, ti02, ti10, ti11, ti12,
     ei00, ei01, ei02, ei03, ei04, ei10, ei11, ei12, ei13, ei14,
     rows, isem0, isem1, gsem0, gsem1, osem0, osem1) = refs
    wid = lax.axis_index("s") * _NC + lax.axis_index("c")
    isems, gsems, osems = (isem0, isem1), (gsem0, gsem1), (osem0, osem1)

    def phase(t_hbm, i_hbms, o_hbm, idxp, chunk, iters, per_w):
        d = len(i_hbms)
        base0 = wid * per_w
        icps, gcps, ocps = {}, {}, {}

        def fire_idx(n):
            p = n % 2
            icps[n] = [pltpu.async_copy(
                i_hbms[j].at[pl.ds(base0 + n * chunk, chunk)], idxp[p][j],
                isems[p]) for j in range(d)]

        def fire_gather(n):
            p = n % 2
            gcps[n] = [pltpu.async_copy(
                t_hbm.at[idxp[p][j]],
                rows.at[pl.ds(p * _HALF + j * chunk, chunk), :], gsems[p])
                for j in range(d)]

        fire_idx(0)
        for cp in icps[0]:
            cp.wait()
        fire_gather(0)
        if iters > 1:
            fire_idx(1)
        for n in range(iters):
            p = n % 2
            if n + 1 < iters:
                for cp in icps[n + 1]:
                    cp.wait()
                if n - 1 >= 0:
                    for cp in ocps[n - 1]:
                        cp.wait()
                fire_gather(n + 1)
            for cp in gcps[n]:
                cp.wait()

            def sum16(t, c):
                for k in range(H // 16):
                    sl = pl.ds(k * 16, 16)
                    s = rows[p * _HALF + t, sl]
                    for j in range(1, d):
                        s = s + rows[p * _HALF + j * chunk + t, sl]
                    rows[p * _HALF + t, sl] = s
                return c

            lax.fori_loop(0, chunk, sum16, 0)
            ocps[n] = [pltpu.async_copy(
                rows.at[pl.ds(p * _HALF, chunk), :],
                o_hbm.at[pl.ds(base0 + n * chunk, chunk), :], osems[p])]
            if n + 2 < iters:
                fire_idx(n + 2)
        for n in (iters - 2, iters - 1):
            if n >= 0:
                for cp in ocps[n]:
                    cp.wait()

    phase(tt_hbm, (it0_h, it1_h, it2_h), ot_hbm,
          ((ti00, ti01, ti02), (ti10, ti11, ti12)),
          _TCHUNK, _TITERS, _TPW)
    phase(te_hbm, (ie0_h, ie1_h, ie2_h, ie3_h, ie4_h), oe_hbm,
          ((ei00, ei01, ei02, ei03, ei04), (ei10, ei11, ei12, ei13, ei14)),
          _ECHUNK, _EITERS, _EPW)


def _sc_gather(*args):
    fn = pl.kernel(
        _sc_body,
        mesh=plsc.VectorSubcoreMesh(core_axis_name="c", subcore_axis_name="s"),
        out_type=[
            jax.ShapeDtypeStruct((BATCH * NTT, H), _F32),
            jax.ShapeDtypeStruct((BATCH * NTE, H), _F32),
        ],
        scratch_types=(
            [pltpu.VMEM((_TCHUNK,), jnp.int32)] * (2 * TDN)
            + [pltpu.VMEM((_ECHUNK,), jnp.int32)] * (2 * EDN)
            + [pltpu.VMEM((2 * _HALF, H), _F32)]
            + [pltpu.SemaphoreType.DMA] * 6
        ),
    )
    return fn(*args)


_TOKT = 25
_NTB = NTT // _TOKT
_FGRID = (NTT + NTE) // _TOKT
_FROWS = _TOKT * BATCH


def _fin_body(gt_ref, ge_ref, ctt_ref, cet_ref, ut_ref, vt_ref, ue_ref,
              ve_ref, out_ref):
    pid = pl.program_id(0)
    dnums = (((0,), (0,)), ((), ()))

    @pl.when(pid < _NTB)
    def _tile():
        out_ref[...] = gt_ref[...] + lax.dot_general(
            ctt_ref[...], ut_ref[...], dnums, preferred_element_type=_F32
        ) + vt_ref[0][None, :]

    @pl.when(pid >= _NTB)
    def _ent():
        out_ref[...] = ge_ref[...] + lax.dot_general(
            cet_ref[...], ue_ref[...], dnums, preferred_element_type=_F32
        ) + ve_ref[0][None, :]


def kernel(tile_continuous, tile_discrete, entity_continuous, entity_discrete,
           entity_N, tile_cont_w, tile_cont_b, tile_table, tile_attr_w,
           tile_attr_b, ent_cont_w, ent_cont_b, ent_table, ent_attr_w,
           ent_attr_b):
    del entity_N
    tt = _transform_table(tile_table, tile_attr_w[TCN * H:, :], TDN)
    te = _transform_table(ent_table, ent_attr_w[ECN * H:, :], EDN)

    it = [(tile_discrete[:, :, j].T.reshape(-1) + j * VOC).astype(jnp.int32)
          for j in range(TDN)]
    ie = [(entity_discrete[:, :, j].T.reshape(-1) + j * VOC).astype(jnp.int32)
          for j in range(EDN)]

    gt, ge = _sc_gather(tt.reshape(TDN * VOC, H), *it,
                        te.reshape(EDN * VOC, H), *ie)

    ut, vt, ue, ve = pl.pallas_call(
        _params_body,
        out_shape=[
            jax.ShapeDtypeStruct((TCN, H), _F32),
            jax.ShapeDtypeStruct((1, H), _F32),
            jax.ShapeDtypeStruct((ECN, H), _F32),
            jax.ShapeDtypeStruct((1, H), _F32),
        ],
    )(tile_cont_w, tile_cont_b, tile_attr_w[:TCN * H, :],
      tile_attr_b.reshape(1, H), ent_cont_w, ent_cont_b,
      ent_attr_w[:ECN * H, :], ent_attr_b.reshape(1, H))

    ctt = jnp.transpose(tile_continuous, (2, 1, 0)).reshape(TCN, NTT * BATCH)
    cet = jnp.transpose(entity_continuous, (2, 1, 0)).reshape(ECN, NTE * BATCH)
    ntb = _NTB
    out_flat = pl.pallas_call(
        _fin_body,
        grid=(_FGRID,),
        in_specs=[
            pl.BlockSpec((_FROWS, H), lambda j: (jnp.minimum(j, ntb - 1), 0)),
            pl.BlockSpec((_FROWS, H), lambda j: (jnp.maximum(j - ntb, 0), 0)),
            pl.BlockSpec((TCN, _FROWS),
                         lambda j: (0, jnp.minimum(j, ntb - 1))),
            pl.BlockSpec((ECN, _FROWS),
                         lambda j: (0, jnp.maximum(j - ntb, 0))),
            pl.BlockSpec((TCN, H), lambda j: (0, 0)),
            pl.BlockSpec((1, H), lambda j: (0, 0)),
            pl.BlockSpec((ECN, H), lambda j: (0, 0)),
            pl.BlockSpec((1, H), lambda j: (0, 0)),
        ],
        out_specs=pl.BlockSpec((_FROWS, H), lambda j: (j, 0)),
        out_shape=jax.ShapeDtypeStruct(((NTT + NTE) * BATCH, H), _F32),
    )(gt, ge, ctt, cet, ut, vt, ue, ve)
    return jnp.transpose(out_flat.reshape(NTT + NTE, BATCH, H), (1, 0, 2))
